# Initial kernel scaffold; baseline (speedup 1.0000x reference)
import jax
import jax.numpy as jnp
from jax.experimental import pallas as pl


def kernel(x, edge_index, eps, W1, W_mu, W_logstd):
    raise NotImplementedError("write your pallas kernel here")


# speedup vs baseline: 4.1962x; 4.1962x over previous
import functools

import jax
import jax.numpy as jnp
from jax import lax
from jax.experimental import pallas as pl
from jax.experimental.pallas import tpu as pltpu
from jax.experimental.pallas import tpu_sc as plsc

N = 10000
NP = 10240
HF = 32
NC, NS = 2, 16
NW = NC * NS
CH = 80
BD = 512


def _segsum_kernel(E):
    epw = E // NW
    nchunk = epw // CH
    rps = NP // NS

    mesh = plsc.VectorSubcoreMesh(
        core_axis_name="c", subcore_axis_name="s", num_cores=NC, num_subcores=NS
    )

    @functools.partial(
        pl.kernel,
        out_type=jax.ShapeDtypeStruct((NC, NP, HF), jnp.float32),
        mesh=mesh,
        scratch_types=[
            pltpu.VMEM((CH,), jnp.int32),
            pltpu.VMEM((CH,), jnp.int32),
            pltpu.VMEM((CH, HF), jnp.float32),
            pltpu.VMEM_SHARED((NP, HF), jnp.float32),
            pltpu.SemaphoreType.DMA,
        ],
        compiler_params=pltpu.CompilerParams(use_tc_tiling_on_sc=False),
    )
    def seg(hw_hbm, src_hbm, dst_hbm, zeros_hbm, out_hbm,
            src_v, dst_v, rows_v, acc_sh, sem):
        c = lax.axis_index("c")
        s = lax.axis_index("s")
        wid = s * NC + c

        pltpu.sync_copy(zeros_hbm.at[pl.ds(s * rps, rps)],
                        acc_sh.at[pl.ds(s * rps, rps)])
        plsc.subcore_barrier()

        base = wid * epw

        def body(i, carry):
            off = pl.multiple_of(base + i * CH, 8)
            pltpu.sync_copy(src_hbm.at[pl.ds(off, CH)], src_v)
            pltpu.sync_copy(dst_hbm.at[pl.ds(off, CH)], dst_v)
            pltpu.async_copy(hw_hbm.at[src_v], rows_v, sem).wait()
            pltpu.sync_copy(rows_v, acc_sh.at[dst_v], add=True)
            return carry

        lax.fori_loop(0, nchunk, body, 0)
        plsc.subcore_barrier()

        pltpu.sync_copy(acc_sh.at[pl.ds(s * rps, rps)],
                        out_hbm.at[c, pl.ds(s * rps, rps)])

    return seg


def _mm1_body(x_ref, w_ref, out_ref):
    out_ref[pl.ds(0, N), :] = jnp.dot(x_ref[...], w_ref[...],
                                      preferred_element_type=jnp.float32)
    out_ref[pl.ds(N, NP - N), :] = jnp.zeros((NP - N, HF), jnp.float32)


def _mm2_body(p_ref, w_ref, out_ref):
    h = jnp.maximum(p_ref[0] + p_ref[1], 0.0)
    out_ref[...] = jnp.dot(h, w_ref[...], preferred_element_type=jnp.float32)


def _z_body(p_ref, eps_ref, out_ref):
    agg = p_ref[0] + p_ref[1]
    out_ref[...] = agg[:, :16] + eps_ref[...] * jnp.exp(agg[:, 16:])


def _dec_body(zi_ref, zj_ref, out_ref):
    out_ref[...] = lax.dot_general(
        zi_ref[...], zj_ref[...], (((1,), (1,)), ((), ())),
        preferred_element_type=jnp.float32)


def kernel(x, edge_index, eps, W1, W_mu, W_logstd):
    E = edge_index.shape[1]
    src = edge_index[0]
    dst = edge_index[1]
    Wc = jnp.concatenate([W_mu, W_logstd], axis=1)
    zeros = jnp.zeros((NP, HF), jnp.float32)
    eps_p = jnp.zeros((NP, 16), jnp.float32).at[:N].set(eps)

    hw1 = pl.pallas_call(
        _mm1_body,
        out_shape=jax.ShapeDtypeStruct((NP, HF), jnp.float32),
    )(x, W1)

    segsum = _segsum_kernel(E)
    p1 = segsum(hw1, src, dst, zeros)

    hw2 = pl.pallas_call(
        _mm2_body,
        out_shape=jax.ShapeDtypeStruct((NP, HF), jnp.float32),
    )(p1, Wc)

    p2 = segsum(hw2, src, dst, zeros)

    zp = pl.pallas_call(
        _z_body,
        out_shape=jax.ShapeDtypeStruct((NP, 16), jnp.float32),
    )(p2, eps_p)

    recon = pl.pallas_call(
        _dec_body,
        grid=(NP // BD, NP // BD),
        in_specs=[
            pl.BlockSpec((BD, 16), lambda i, j: (i, 0)),
            pl.BlockSpec((BD, 16), lambda i, j: (j, 0)),
        ],
        out_specs=pl.BlockSpec((BD, BD), lambda i, j: (i, j)),
        out_shape=jax.ShapeDtypeStruct((N, N), jnp.float32),
    )(zp, zp)

    return recon.reshape(-1)


# speedup vs baseline: 7.5815x; 1.8068x over previous
import functools

import jax
import jax.numpy as jnp
from jax import lax
from jax.experimental import pallas as pl
from jax.experimental.pallas import tpu as pltpu
from jax.experimental.pallas import tpu_sc as plsc

N = 10000
NP = 10240
HF = 32
NC, NS = 2, 16
NW = NC * NS
CH = 80
BD = 512
BDJ = 2560


NBUF = 5


def _segsum_kernel(E):
    epw = E // NW
    nchunk = epw // CH
    cpw = nchunk
    rps = NP // NS

    mesh = plsc.VectorSubcoreMesh(
        core_axis_name="c", subcore_axis_name="s", num_cores=NC, num_subcores=NS
    )

    @functools.partial(
        pl.kernel,
        out_type=jax.ShapeDtypeStruct((NC, NP, HF), jnp.float32),
        mesh=mesh,
        scratch_types=[
            pltpu.VMEM((cpw, CH), jnp.int32),
            pltpu.VMEM((cpw, CH), jnp.int32),
            [pltpu.VMEM((CH, HF), jnp.float32) for _ in range(NBUF)],
            pltpu.VMEM_SHARED((NP, HF), jnp.float32),
            [pltpu.SemaphoreType.DMA for _ in range(NBUF)],
            pltpu.SemaphoreType.DMA,
        ],
        compiler_params=pltpu.CompilerParams(use_tc_tiling_on_sc=False),
    )
    def seg(hw_hbm, src2_hbm, dst2_hbm, zeros_hbm, out_hbm,
            src_v, dst_v, rows_v, acc_sh, gsem, isem):
        c = lax.axis_index("c")
        s = lax.axis_index("s")
        wid = s * NC + c
        cbase = wid * cpw

        pltpu.async_copy(src2_hbm.at[pl.ds(cbase, cpw)], src_v, isem)
        pltpu.async_copy(dst2_hbm.at[pl.ds(cbase, cpw)], dst_v, isem)
        pltpu.make_async_copy(src2_hbm.at[pl.ds(cbase, cpw)], src_v, isem).wait()
        pltpu.make_async_copy(dst2_hbm.at[pl.ds(cbase, cpw)], dst_v, isem).wait()

        for b in range(NBUF):
            pltpu.async_copy(hw_hbm.at[src_v.at[b]], rows_v[b], gsem[b])

        pltpu.sync_copy(zeros_hbm.at[pl.ds(s * rps, rps)],
                        acc_sh.at[pl.ds(s * rps, rps)])
        plsc.subcore_barrier()

        def outer(io, carry):
            i0 = io * NBUF
            for b in range(NBUF):
                i = i0 + b
                pltpu.make_async_copy(hw_hbm.at[src_v.at[i]], rows_v[b],
                                      gsem[b]).wait()
                pltpu.sync_copy(rows_v[b], acc_sh.at[dst_v.at[i]], add=True)
                nxt = i + NBUF

                @pl.when(nxt < nchunk)
                def _():
                    pltpu.async_copy(hw_hbm.at[src_v.at[nxt]], rows_v[b],
                                     gsem[b])
            return carry

        lax.fori_loop(0, nchunk // NBUF, outer, 0)
        plsc.subcore_barrier()

        pltpu.sync_copy(acc_sh.at[pl.ds(s * rps, rps)],
                        out_hbm.at[c, pl.ds(s * rps, rps)])

    return seg


def _mm1_body(x_ref, w_ref, out_ref):
    out_ref[pl.ds(0, N), :] = jnp.dot(x_ref[...], w_ref[...],
                                      preferred_element_type=jnp.float32)
    out_ref[pl.ds(N, NP - N), :] = jnp.zeros((NP - N, HF), jnp.float32)


def _mm2_body(p_ref, w_ref, out_ref):
    h = jnp.maximum(p_ref[0] + p_ref[1], 0.0)
    out_ref[...] = jnp.dot(h, w_ref[...], preferred_element_type=jnp.float32)


def _z_body(p_ref, eps_ref, out_ref):
    agg = p_ref[0] + p_ref[1]
    out_ref[...] = agg[:, :16] + eps_ref[...] * jnp.exp(agg[:, 16:])


def _dec_body(zi_ref, zj_ref, out_ref):
    out_ref[...] = lax.dot_general(
        zi_ref[...], zj_ref[...], (((1,), (1,)), ((), ())),
        preferred_element_type=jnp.float32)


def kernel(x, edge_index, eps, W1, W_mu, W_logstd):
    E = edge_index.shape[1]
    src = edge_index[0].reshape(E // CH, CH)
    dst = edge_index[1].reshape(E // CH, CH)
    Wc = jnp.concatenate([W_mu, W_logstd], axis=1)
    zeros = jnp.zeros((NP, HF), jnp.float32)
    eps_p = jnp.zeros((NP, 16), jnp.float32).at[:N].set(eps)

    hw1 = pl.pallas_call(
        _mm1_body,
        out_shape=jax.ShapeDtypeStruct((NP, HF), jnp.float32),
    )(x, W1)

    segsum = _segsum_kernel(E)
    p1 = segsum(hw1, src, dst, zeros)

    hw2 = pl.pallas_call(
        _mm2_body,
        out_shape=jax.ShapeDtypeStruct((NP, HF), jnp.float32),
    )(p1, Wc)

    p2 = segsum(hw2, src, dst, zeros)

    zp = pl.pallas_call(
        _z_body,
        out_shape=jax.ShapeDtypeStruct((NP, 16), jnp.float32),
    )(p2, eps_p)

    recon = pl.pallas_call(
        _dec_body,
        grid=(NP // BD, NP // BDJ),
        in_specs=[
            pl.BlockSpec((BD, 16), lambda i, j: (i, 0)),
            pl.BlockSpec((BDJ, 16), lambda i, j: (j, 0)),
        ],
        out_specs=pl.BlockSpec((BD, BDJ), lambda i, j: (i, j)),
        out_shape=jax.ShapeDtypeStruct((N, N), jnp.float32),
    )(zp, zp)

    return recon.reshape(-1)


# speedup vs baseline: 13.8821x; 1.8311x over previous
import functools

import jax
import jax.numpy as jnp
from jax import lax
from jax.experimental import pallas as pl
from jax.experimental.pallas import tpu as pltpu
from jax.experimental.pallas import tpu_sc as plsc

N = 10000
NP = 10240
HF = 32
NC, NS = 2, 16
NW = NC * NS
CH = 80
BD = 512
BDJ = 2560


NBUF = 5


def _segsum_kernel(E):
    epw = E // NW
    nchunk = epw // CH
    cpw = nchunk
    rps = NP // NS

    mesh = plsc.VectorSubcoreMesh(
        core_axis_name="c", subcore_axis_name="s", num_cores=NC, num_subcores=NS
    )

    @functools.partial(
        pl.kernel,
        out_type=jax.ShapeDtypeStruct((NC, NP, HF), jnp.float32),
        mesh=mesh,
        scratch_types=[
            pltpu.VMEM((cpw, CH), jnp.int32),
            pltpu.VMEM((cpw, CH), jnp.int32),
            [pltpu.VMEM((CH, HF), jnp.float32) for _ in range(NBUF)],
            pltpu.VMEM_SHARED((NP, HF), jnp.float32),
            [pltpu.SemaphoreType.DMA for _ in range(NBUF)],
            pltpu.SemaphoreType.DMA,
        ],
        compiler_params=pltpu.CompilerParams(use_tc_tiling_on_sc=False),
    )
    def seg(hw_hbm, src2_hbm, dst2_hbm, zeros_hbm, out_hbm,
            src_v, dst_v, rows_v, acc_sh, gsem, isem):
        c = lax.axis_index("c")
        s = lax.axis_index("s")
        wid = s * NC + c
        cbase = wid * cpw

        pltpu.async_copy(src2_hbm.at[pl.ds(cbase, cpw)], src_v, isem)
        pltpu.async_copy(dst2_hbm.at[pl.ds(cbase, cpw)], dst_v, isem)
        pltpu.make_async_copy(src2_hbm.at[pl.ds(cbase, cpw)], src_v, isem).wait()
        pltpu.make_async_copy(dst2_hbm.at[pl.ds(cbase, cpw)], dst_v, isem).wait()

        for b in range(NBUF):
            pltpu.async_copy(hw_hbm.at[src_v.at[b]], rows_v[b], gsem[b])

        pltpu.sync_copy(zeros_hbm.at[pl.ds(s * rps, rps)],
                        acc_sh.at[pl.ds(s * rps, rps)])
        plsc.subcore_barrier()

        def outer(io, carry):
            i0 = io * NBUF
            for b in range(NBUF):
                i = i0 + b
                pltpu.make_async_copy(hw_hbm.at[src_v.at[i]], rows_v[b],
                                      gsem[b]).wait()
                pltpu.sync_copy(rows_v[b], acc_sh.at[dst_v.at[i]], add=True)
                nxt = i + NBUF

                @pl.when(nxt < nchunk)
                def _():
                    pltpu.async_copy(hw_hbm.at[src_v.at[nxt]], rows_v[b],
                                     gsem[b])
            return carry

        lax.fori_loop(0, nchunk // NBUF, outer, 0)
        plsc.subcore_barrier()

        pltpu.sync_copy(acc_sh.at[pl.ds(s * rps, rps)],
                        out_hbm.at[c, pl.ds(s * rps, rps)])

    return seg


def _mm1_body(x_ref, w_ref, out_ref):
    out_ref[pl.ds(0, N), :] = jnp.dot(x_ref[...], w_ref[...],
                                      preferred_element_type=jnp.float32)
    out_ref[pl.ds(N, NP - N), :] = jnp.zeros((NP - N, HF), jnp.float32)


def _mm2_body(p_ref, w_ref, out_ref):
    h = jnp.maximum(p_ref[0] + p_ref[1], 0.0)
    out_ref[...] = jnp.dot(h, w_ref[...], preferred_element_type=jnp.float32)


def _z_body(p_ref, eps_ref, out_ref):
    agg = p_ref[0] + p_ref[1]
    out_ref[...] = agg[:, :16] + eps_ref[...] * jnp.exp(agg[:, 16:])


SR = 64


def _dec_body(zi_ref, zj_ref, out_ref):
    res = lax.dot_general(
        zi_ref[...], zj_ref[...], (((1,), (1,)), ((), ())),
        preferred_element_type=jnp.float32)
    for r in range(SR):
        out_ref[pl.ds(r * N, N)] = res[r, :N]


def kernel(x, edge_index, eps, W1, W_mu, W_logstd):
    E = edge_index.shape[1]
    src = edge_index[0].reshape(E // CH, CH)
    dst = edge_index[1].reshape(E // CH, CH)
    Wc = jnp.concatenate([W_mu, W_logstd], axis=1)
    zeros = jnp.zeros((NP, HF), jnp.float32)
    eps_p = jnp.zeros((NP, 16), jnp.float32).at[:N].set(eps)

    hw1 = pl.pallas_call(
        _mm1_body,
        out_shape=jax.ShapeDtypeStruct((NP, HF), jnp.float32),
    )(x, W1)

    segsum = _segsum_kernel(E)
    p1 = segsum(hw1, src, dst, zeros)

    hw2 = pl.pallas_call(
        _mm2_body,
        out_shape=jax.ShapeDtypeStruct((NP, HF), jnp.float32),
    )(p1, Wc)

    p2 = segsum(hw2, src, dst, zeros)

    zp = pl.pallas_call(
        _z_body,
        out_shape=jax.ShapeDtypeStruct((NP, 16), jnp.float32),
    )(p2, eps_p)

    recon = pl.pallas_call(
        _dec_body,
        grid=((N + SR - 1) // SR,),
        in_specs=[
            pl.BlockSpec((SR, 16), lambda i: (i, 0)),
            pl.BlockSpec((NP, 16), lambda i: (0, 0)),
        ],
        out_specs=pl.BlockSpec((SR * N,), lambda i: (i,)),
        out_shape=jax.ShapeDtypeStruct((N * N,), jnp.float32),
    )(zp, zp)

    return recon


# speedup vs baseline: 15.4474x; 1.1128x over previous
import functools

import jax
import jax.numpy as jnp
from jax import lax
from jax.experimental import pallas as pl
from jax.experimental.pallas import tpu as pltpu
from jax.experimental.pallas import tpu_sc as plsc

N = 10000
NP = 10240
HF = 32
NC, NS = 2, 16
NW = NC * NS
CH = 80
BD = 512
BDJ = 2560


NBUF = 5


def _segsum_kernel(E):
    epw = E // NW
    nchunk = epw // CH
    cpw = nchunk
    rps = NP // NS

    mesh = plsc.VectorSubcoreMesh(
        core_axis_name="c", subcore_axis_name="s", num_cores=NC, num_subcores=NS
    )

    @functools.partial(
        pl.kernel,
        out_type=jax.ShapeDtypeStruct((NC, NP, HF), jnp.float32),
        mesh=mesh,
        scratch_types=[
            pltpu.VMEM((cpw, CH), jnp.int32),
            pltpu.VMEM((cpw, CH), jnp.int32),
            [pltpu.VMEM((CH, HF), jnp.float32) for _ in range(NBUF)],
            pltpu.VMEM_SHARED((NP, HF), jnp.float32),
            [pltpu.SemaphoreType.DMA for _ in range(NBUF)],
            pltpu.SemaphoreType.DMA,
        ],
        compiler_params=pltpu.CompilerParams(use_tc_tiling_on_sc=False),
    )
    def seg(hw_hbm, src2_hbm, dst2_hbm, zeros_hbm, out_hbm,
            src_v, dst_v, rows_v, acc_sh, gsem, isem):
        c = lax.axis_index("c")
        s = lax.axis_index("s")
        wid = s * NC + c
        cbase = wid * cpw

        pltpu.async_copy(src2_hbm.at[pl.ds(cbase, cpw)], src_v, isem)
        pltpu.async_copy(dst2_hbm.at[pl.ds(cbase, cpw)], dst_v, isem)
        pltpu.make_async_copy(src2_hbm.at[pl.ds(cbase, cpw)], src_v, isem).wait()
        pltpu.make_async_copy(dst2_hbm.at[pl.ds(cbase, cpw)], dst_v, isem).wait()

        for b in range(NBUF):
            pltpu.async_copy(hw_hbm.at[src_v.at[b]], rows_v[b], gsem[b])

        pltpu.sync_copy(zeros_hbm.at[pl.ds(s * rps, rps)],
                        acc_sh.at[pl.ds(s * rps, rps)])
        plsc.subcore_barrier()

        def outer(io, carry):
            i0 = io * NBUF
            for b in range(NBUF):
                i = i0 + b
                pltpu.make_async_copy(hw_hbm.at[src_v.at[i]], rows_v[b],
                                      gsem[b]).wait()
                pltpu.sync_copy(rows_v[b], acc_sh.at[dst_v.at[i]], add=True)
                nxt = i + NBUF

                @pl.when(nxt < nchunk)
                def _():
                    pltpu.async_copy(hw_hbm.at[src_v.at[nxt]], rows_v[b],
                                     gsem[b])
            return carry

        lax.fori_loop(0, nchunk // NBUF, outer, 0)
        plsc.subcore_barrier()

        pltpu.sync_copy(acc_sh.at[pl.ds(s * rps, rps)],
                        out_hbm.at[c, pl.ds(s * rps, rps)])

    return seg


def _mm1_body(x_ref, w_ref, out_ref):
    out_ref[pl.ds(0, N), :] = jnp.dot(x_ref[...], w_ref[...],
                                      preferred_element_type=jnp.float32)
    out_ref[pl.ds(N, NP - N), :] = jnp.zeros((NP - N, HF), jnp.float32)


def _mm2_body(p_ref, w_ref, out_ref):
    h = jnp.maximum(p_ref[0] + p_ref[1], 0.0)
    out_ref[...] = jnp.dot(h, w_ref[...], preferred_element_type=jnp.float32)


def _z_body(p_ref, eps_ref, out_ref):
    agg = p_ref[0] + p_ref[1]
    out_ref[...] = agg[:, :16] + eps_ref[...] * jnp.exp(agg[:, 16:])


SR = 128


def _dec_body(zi_ref, zj_ref, out_ref):
    res = lax.dot_general(
        zi_ref[...], zj_ref[...], (((1,), (1,)), ((), ())),
        preferred_element_type=jnp.float32)
    for r in range(SR):
        out_ref[pl.ds(r * N, N)] = res[r, :N]


def kernel(x, edge_index, eps, W1, W_mu, W_logstd):
    E = edge_index.shape[1]
    src = edge_index[0].reshape(E // CH, CH)
    dst = edge_index[1].reshape(E // CH, CH)
    Wc = jnp.concatenate([W_mu, W_logstd], axis=1)
    zeros = jnp.zeros((NP, HF), jnp.float32)
    eps_p = jnp.zeros((NP, 16), jnp.float32).at[:N].set(eps)

    hw1 = pl.pallas_call(
        _mm1_body,
        out_shape=jax.ShapeDtypeStruct((NP, HF), jnp.float32),
    )(x, W1)

    segsum = _segsum_kernel(E)
    p1 = segsum(hw1, src, dst, zeros)

    hw2 = pl.pallas_call(
        _mm2_body,
        out_shape=jax.ShapeDtypeStruct((NP, HF), jnp.float32),
    )(p1, Wc)

    p2 = segsum(hw2, src, dst, zeros)

    zp = pl.pallas_call(
        _z_body,
        out_shape=jax.ShapeDtypeStruct((NP, 16), jnp.float32),
    )(p2, eps_p)

    recon = pl.pallas_call(
        _dec_body,
        grid=((N + SR - 1) // SR,),
        in_specs=[
            pl.BlockSpec((SR, 16), lambda i: (i, 0)),
            pl.BlockSpec((NP, 16), lambda i: (0, 0)),
        ],
        out_specs=pl.BlockSpec((SR * N,), lambda i: (i,)),
        out_shape=jax.ShapeDtypeStruct((N * N,), jnp.float32),
    )(zp, zp)

    return recon


# speedup vs baseline: 16.2837x; 1.0541x over previous
import functools

import jax
import jax.numpy as jnp
from jax import lax
from jax.experimental import pallas as pl
from jax.experimental.pallas import tpu as pltpu
from jax.experimental.pallas import tpu_sc as plsc

N = 10000
NP = 10240
HF = 32
NC, NS = 2, 16
NW = NC * NS
CH = 80
BD = 512
BDJ = 2560


NBUF = 5


def _segsum_kernel(E):
    epw = E // NW
    nchunk = epw // CH
    cpw = nchunk
    rps = NP // NS

    mesh = plsc.VectorSubcoreMesh(
        core_axis_name="c", subcore_axis_name="s", num_cores=NC, num_subcores=NS
    )

    @functools.partial(
        pl.kernel,
        out_type=jax.ShapeDtypeStruct((NC, NP, HF), jnp.float32),
        mesh=mesh,
        scratch_types=[
            pltpu.VMEM((cpw, CH), jnp.int32),
            pltpu.VMEM((cpw, CH), jnp.int32),
            [pltpu.VMEM((CH, HF), jnp.float32) for _ in range(NBUF)],
            pltpu.VMEM_SHARED((NP, HF), jnp.float32),
            [pltpu.SemaphoreType.DMA for _ in range(NBUF)],
            pltpu.SemaphoreType.DMA,
        ],
        compiler_params=pltpu.CompilerParams(use_tc_tiling_on_sc=False),
    )
    def seg(hw_hbm, src2_hbm, dst2_hbm, zeros_hbm, out_hbm,
            src_v, dst_v, rows_v, acc_sh, gsem, isem):
        c = lax.axis_index("c")
        s = lax.axis_index("s")
        wid = s * NC + c
        cbase = wid * cpw

        pltpu.async_copy(src2_hbm.at[pl.ds(cbase, cpw)], src_v, isem)
        pltpu.async_copy(dst2_hbm.at[pl.ds(cbase, cpw)], dst_v, isem)
        pltpu.make_async_copy(src2_hbm.at[pl.ds(cbase, cpw)], src_v, isem).wait()
        pltpu.make_async_copy(dst2_hbm.at[pl.ds(cbase, cpw)], dst_v, isem).wait()

        for b in range(NBUF):
            pltpu.async_copy(hw_hbm.at[src_v.at[b]], rows_v[b], gsem[b])

        pltpu.sync_copy(zeros_hbm.at[pl.ds(s * rps, rps)],
                        acc_sh.at[pl.ds(s * rps, rps)])
        plsc.subcore_barrier()

        def outer(io, carry):
            i0 = io * NBUF
            for b in range(NBUF):
                i = i0 + b
                pltpu.make_async_copy(hw_hbm.at[src_v.at[i]], rows_v[b],
                                      gsem[b]).wait()
                pltpu.sync_copy(rows_v[b], acc_sh.at[dst_v.at[i]], add=True)
                nxt = i + NBUF

                @pl.when(nxt < nchunk)
                def _():
                    pltpu.async_copy(hw_hbm.at[src_v.at[nxt]], rows_v[b],
                                     gsem[b])
            return carry

        lax.fori_loop(0, nchunk // NBUF, outer, 0)
        plsc.subcore_barrier()

        pltpu.sync_copy(acc_sh.at[pl.ds(s * rps, rps)],
                        out_hbm.at[c, pl.ds(s * rps, rps)])

    return seg


NPK = NP * HF // 128


def _mm1_body(x_ref, w_ref, out_ref):
    out_ref[pl.ds(0, N), :] = jnp.dot(x_ref[...], w_ref[...],
                                      preferred_element_type=jnp.float32)
    out_ref[pl.ds(N, NP - N), :] = jnp.zeros((NP - N, HF), jnp.float32)


def _mm2_body(p_ref, w_ref, out_ref):
    h = jnp.maximum(p_ref[0] + p_ref[1], 0.0)
    out_ref[...] = jnp.dot(h, w_ref[...], preferred_element_type=jnp.float32)


def _z_body(p_ref, eps_ref, out_ref):
    agg = p_ref[0] + p_ref[1]
    z = agg[:N, :16] + eps_ref[...] * jnp.exp(agg[:N, 16:])
    out_ref[pl.ds(0, N), :] = z
    out_ref[pl.ds(N, NP - N), :] = jnp.zeros((NP - N, 16), jnp.float32)


SR = 128


def _dec_body(zi_ref, zj_ref, out_ref):
    res = lax.dot_general(
        zi_ref[...], zj_ref[...], (((1,), (1,)), ((), ())),
        preferred_element_type=jnp.float32)
    for r in range(SR):
        out_ref[pl.ds(r * N, N)] = res[r, :N]


def kernel(x, edge_index, eps, W1, W_mu, W_logstd):
    E = edge_index.shape[1]
    src = edge_index[0].reshape(E // CH, CH)
    dst = edge_index[1].reshape(E // CH, CH)
    Wc = jnp.concatenate([W_mu, W_logstd], axis=1)
    Wbig = jnp.kron(jnp.eye(4, dtype=jnp.float32), Wc)
    zeros = jnp.zeros((NP, HF), jnp.float32)

    hw1 = pl.pallas_call(
        _mm1_body,
        out_shape=jax.ShapeDtypeStruct((NP, HF), jnp.float32),
    )(x, W1)

    segsum = _segsum_kernel(E)
    p1 = segsum(hw1, src, dst, zeros)

    hw2 = pl.pallas_call(
        _mm2_body,
        out_shape=jax.ShapeDtypeStruct((NPK, 128), jnp.float32),
    )(p1.reshape(NC, NPK, 128), Wbig)

    p2 = segsum(hw2.reshape(NP, HF), src, dst, zeros)

    zp = pl.pallas_call(
        _z_body,
        out_shape=jax.ShapeDtypeStruct((NP, 16), jnp.float32),
    )(p2, eps)

    recon = pl.pallas_call(
        _dec_body,
        grid=((N + SR - 1) // SR,),
        in_specs=[
            pl.BlockSpec((SR, 16), lambda i: (i, 0)),
            pl.BlockSpec((NP, 16), lambda i: (0, 0)),
        ],
        out_specs=pl.BlockSpec((SR * N,), lambda i: (i,)),
        out_shape=jax.ShapeDtypeStruct((N * N,), jnp.float32),
    )(zp, zp)

    return recon


# speedup vs baseline: 16.7166x; 1.0266x over previous
import functools

import jax
import jax.numpy as jnp
from jax import lax
from jax.experimental import pallas as pl
from jax.experimental.pallas import tpu as pltpu
from jax.experimental.pallas import tpu_sc as plsc

N = 10000
NP = 10240
HF = 32
NC, NS = 2, 16
NW = NC * NS
CH = 128
NBUF = 6


def _segsum_kernel(E):
    nrows = E // CH
    cpw = nrows // NW
    ntail = nrows - cpw * NW
    rps = NP // NS

    mesh = plsc.VectorSubcoreMesh(
        core_axis_name="c", subcore_axis_name="s", num_cores=NC, num_subcores=NS
    )

    @functools.partial(
        pl.kernel,
        out_type=jax.ShapeDtypeStruct((NC, NP, HF), jnp.float32),
        mesh=mesh,
        scratch_types=[
            pltpu.VMEM((cpw + 1, CH), jnp.int32),
            pltpu.VMEM((cpw + 1, CH), jnp.int32),
            [pltpu.VMEM((CH, HF), jnp.float32) for _ in range(NBUF)],
            pltpu.VMEM_SHARED((NP, HF), jnp.float32),
            [pltpu.SemaphoreType.DMA for _ in range(NBUF)],
            pltpu.SemaphoreType.DMA,
        ],
        compiler_params=pltpu.CompilerParams(use_tc_tiling_on_sc=False),
    )
    def seg(hw_hbm, src2_hbm, dst2_hbm, zeros_hbm, out_hbm,
            src_v, dst_v, rows_v, acc_sh, gsem, isem):
        c = lax.axis_index("c")
        s = lax.axis_index("s")
        wid = s * NC + c
        cbase = wid * cpw
        has_tail = wid < ntail

        pltpu.async_copy(src2_hbm.at[pl.ds(cbase, cpw)],
                         src_v.at[pl.ds(0, cpw)], isem)
        pltpu.async_copy(dst2_hbm.at[pl.ds(cbase, cpw)],
                         dst_v.at[pl.ds(0, cpw)], isem)

        @pl.when(has_tail)
        def _():
            t = NW * cpw + wid
            pltpu.async_copy(src2_hbm.at[pl.ds(t, 1)],
                             src_v.at[pl.ds(cpw, 1)], isem)
            pltpu.async_copy(dst2_hbm.at[pl.ds(t, 1)],
                             dst_v.at[pl.ds(cpw, 1)], isem)
            pltpu.make_async_copy(src2_hbm.at[pl.ds(t, 1)],
                                  src_v.at[pl.ds(cpw, 1)], isem).wait()
            pltpu.make_async_copy(dst2_hbm.at[pl.ds(t, 1)],
                                  dst_v.at[pl.ds(cpw, 1)], isem).wait()

        pltpu.make_async_copy(src2_hbm.at[pl.ds(cbase, cpw)],
                              src_v.at[pl.ds(0, cpw)], isem).wait()
        pltpu.make_async_copy(dst2_hbm.at[pl.ds(cbase, cpw)],
                              dst_v.at[pl.ds(0, cpw)], isem).wait()

        for b in range(NBUF):
            pltpu.async_copy(hw_hbm.at[src_v.at[b]], rows_v[b], gsem[b])

        pltpu.sync_copy(zeros_hbm.at[pl.ds(s * rps, rps)],
                        acc_sh.at[pl.ds(s * rps, rps)])
        plsc.subcore_barrier()

        def outer(io, carry):
            i0 = io * NBUF
            for b in range(NBUF):
                i = i0 + b
                pltpu.make_async_copy(hw_hbm.at[src_v.at[i]], rows_v[b],
                                      gsem[b]).wait()
                pltpu.sync_copy(rows_v[b], acc_sh.at[dst_v.at[i]], add=True)
                nxt = i + NBUF

                @pl.when(nxt < cpw)
                def _():
                    pltpu.async_copy(hw_hbm.at[src_v.at[nxt]], rows_v[b],
                                     gsem[b])
            return carry

        lax.fori_loop(0, cpw // NBUF, outer, 0)

        @pl.when(has_tail)
        def _():
            pltpu.async_copy(hw_hbm.at[src_v.at[cpw]], rows_v[0], gsem[0])
            pltpu.make_async_copy(hw_hbm.at[src_v.at[cpw]], rows_v[0],
                                  gsem[0]).wait()
            pltpu.sync_copy(rows_v[0], acc_sh.at[dst_v.at[cpw]], add=True)

        plsc.subcore_barrier()

        pltpu.sync_copy(acc_sh.at[pl.ds(s * rps, rps)],
                        out_hbm.at[c, pl.ds(s * rps, rps)])

    return seg


NPK = NP * HF // 128


def _mm1_body(x_ref, w_ref, out_ref):
    out_ref[pl.ds(0, N), :] = jnp.dot(x_ref[...], w_ref[...],
                                      preferred_element_type=jnp.float32)
    out_ref[pl.ds(N, NP - N), :] = jnp.zeros((NP - N, HF), jnp.float32)


def _mm2_body(p_ref, w_ref, out_ref):
    h = jnp.maximum(p_ref[0] + p_ref[1], 0.0)
    out_ref[...] = jnp.dot(h, w_ref[...], preferred_element_type=jnp.float32)


def _z_body(p_ref, eps_ref, out_ref):
    agg = p_ref[0] + p_ref[1]
    z = agg[:N, :16] + eps_ref[...] * jnp.exp(agg[:N, 16:])
    out_ref[pl.ds(0, N), :] = z
    out_ref[pl.ds(N, NP - N), :] = jnp.zeros((NP - N, 16), jnp.float32)


SR = 128


def _dec_body(zi_ref, zj_ref, out_ref):
    res = lax.dot_general(
        zi_ref[...], zj_ref[...], (((1,), (1,)), ((), ())),
        preferred_element_type=jnp.float32)
    for r in range(SR):
        out_ref[pl.ds(r * N, N)] = res[r, :N]


def kernel(x, edge_index, eps, W1, W_mu, W_logstd):
    E = edge_index.shape[1]
    src = edge_index[0].reshape(E // CH, CH)
    dst = edge_index[1].reshape(E // CH, CH)
    Wc = jnp.concatenate([W_mu, W_logstd], axis=1)
    Wbig = jnp.kron(jnp.eye(4, dtype=jnp.float32), Wc)
    zeros = jnp.zeros((NP, HF), jnp.float32)

    hw1 = pl.pallas_call(
        _mm1_body,
        out_shape=jax.ShapeDtypeStruct((NP, HF), jnp.float32),
    )(x, W1)

    segsum = _segsum_kernel(E)
    p1 = segsum(hw1, src, dst, zeros)

    hw2 = pl.pallas_call(
        _mm2_body,
        out_shape=jax.ShapeDtypeStruct((NPK, 128), jnp.float32),
    )(p1.reshape(NC, NPK, 128), Wbig)

    p2 = segsum(hw2.reshape(NP, HF), src, dst, zeros)

    zp = pl.pallas_call(
        _z_body,
        out_shape=jax.ShapeDtypeStruct((NP, 16), jnp.float32),
    )(p2, eps)

    recon = pl.pallas_call(
        _dec_body,
        grid=((N + SR - 1) // SR,),
        in_specs=[
            pl.BlockSpec((SR, 16), lambda i: (i, 0)),
            pl.BlockSpec((NP, 16), lambda i: (0, 0)),
        ],
        out_specs=pl.BlockSpec((SR * N,), lambda i: (i,)),
        out_shape=jax.ShapeDtypeStruct((N * N,), jnp.float32),
    )(zp, zp)

    return recon


# speedup vs baseline: 17.3458x; 1.0376x over previous
import functools

import jax
import jax.numpy as jnp
from jax import lax
from jax.experimental import pallas as pl
from jax.experimental.pallas import tpu as pltpu
from jax.experimental.pallas import tpu_sc as plsc

N = 10000
NP = 10240
HF = 32
NC, NS = 2, 16
NW = NC * NS
CH = 128
NBUF = 6


def _segsum_kernel(E):
    nrows = E // CH
    cpw = nrows // NW
    ntail = nrows - cpw * NW
    rps = NP // NS

    mesh = plsc.VectorSubcoreMesh(
        core_axis_name="c", subcore_axis_name="s", num_cores=NC, num_subcores=NS
    )

    @functools.partial(
        pl.kernel,
        out_type=jax.ShapeDtypeStruct((NC, NP, HF), jnp.float32),
        mesh=mesh,
        scratch_types=[
            pltpu.VMEM((cpw + 1, CH), jnp.int32),
            pltpu.VMEM((cpw + 1, CH), jnp.int32),
            [pltpu.VMEM((CH, HF), jnp.float32) for _ in range(NBUF)],
            pltpu.VMEM_SHARED((NP, HF), jnp.float32),
            [pltpu.SemaphoreType.DMA for _ in range(NBUF)],
            pltpu.SemaphoreType.DMA,
        ],
        compiler_params=pltpu.CompilerParams(use_tc_tiling_on_sc=False),
    )
    def seg(hw_hbm, src2_hbm, dst2_hbm, zeros_hbm, out_hbm,
            src_v, dst_v, rows_v, acc_sh, gsem, isem):
        c = lax.axis_index("c")
        s = lax.axis_index("s")
        wid = s * NC + c
        cbase = wid * cpw
        has_tail = wid < ntail

        pltpu.async_copy(src2_hbm.at[pl.ds(cbase, cpw)],
                         src_v.at[pl.ds(0, cpw)], isem)
        pltpu.async_copy(dst2_hbm.at[pl.ds(cbase, cpw)],
                         dst_v.at[pl.ds(0, cpw)], isem)

        @pl.when(has_tail)
        def _():
            t = NW * cpw + wid
            pltpu.async_copy(src2_hbm.at[pl.ds(t, 1)],
                             src_v.at[pl.ds(cpw, 1)], isem)
            pltpu.async_copy(dst2_hbm.at[pl.ds(t, 1)],
                             dst_v.at[pl.ds(cpw, 1)], isem)
            pltpu.make_async_copy(src2_hbm.at[pl.ds(t, 1)],
                                  src_v.at[pl.ds(cpw, 1)], isem).wait()
            pltpu.make_async_copy(dst2_hbm.at[pl.ds(t, 1)],
                                  dst_v.at[pl.ds(cpw, 1)], isem).wait()

        pltpu.make_async_copy(src2_hbm.at[pl.ds(cbase, cpw)],
                              src_v.at[pl.ds(0, cpw)], isem).wait()
        pltpu.make_async_copy(dst2_hbm.at[pl.ds(cbase, cpw)],
                              dst_v.at[pl.ds(0, cpw)], isem).wait()

        for b in range(NBUF):
            pltpu.async_copy(hw_hbm.at[src_v.at[b]], rows_v[b], gsem[b])

        pltpu.sync_copy(zeros_hbm.at[pl.ds(s * rps, rps)],
                        acc_sh.at[pl.ds(s * rps, rps)])
        plsc.subcore_barrier()

        def outer(io, carry):
            i0 = io * NBUF
            for b in range(NBUF):
                i = i0 + b
                pltpu.make_async_copy(hw_hbm.at[src_v.at[i]], rows_v[b],
                                      gsem[b]).wait()
                pltpu.sync_copy(rows_v[b], acc_sh.at[dst_v.at[i]], add=True)
                nxt = i + NBUF

                @pl.when(nxt < cpw)
                def _():
                    pltpu.async_copy(hw_hbm.at[src_v.at[nxt]], rows_v[b],
                                     gsem[b])
            return carry

        lax.fori_loop(0, cpw // NBUF, outer, 0)

        @pl.when(has_tail)
        def _():
            pltpu.async_copy(hw_hbm.at[src_v.at[cpw]], rows_v[0], gsem[0])
            pltpu.make_async_copy(hw_hbm.at[src_v.at[cpw]], rows_v[0],
                                  gsem[0]).wait()
            pltpu.sync_copy(rows_v[0], acc_sh.at[dst_v.at[cpw]], add=True)

        plsc.subcore_barrier()

        pltpu.sync_copy(acc_sh.at[pl.ds(s * rps, rps)],
                        out_hbm.at[c, pl.ds(s * rps, rps)])

    return seg


NPK = NP * HF // 128


def _mm1_body(x_ref, w_ref, out_ref):
    out_ref[pl.ds(0, N), :] = jnp.dot(x_ref[...], w_ref[...],
                                      preferred_element_type=jnp.float32)
    out_ref[pl.ds(N, NP - N), :] = jnp.zeros((NP - N, HF), jnp.float32)


def _mm2_body(p_ref, w_ref, out_ref):
    h = jnp.maximum(p_ref[0] + p_ref[1], 0.0)
    out_ref[...] = jnp.dot(h, w_ref[...], preferred_element_type=jnp.float32)


def _z_body(p_ref, eps_ref, out_ref):
    agg = p_ref[0] + p_ref[1]
    z = agg[:N, :16] + eps_ref[...] * jnp.exp(agg[:N, 16:])
    out_ref[pl.ds(0, N), :] = z
    out_ref[pl.ds(N, NP - N), :] = jnp.zeros((NP - N, 16), jnp.float32)


SR = 256


def _dec_body(zi_ref, zj_ref, out_ref):
    res = lax.dot_general(
        zi_ref[...], zj_ref[...], (((1,), (1,)), ((), ())),
        preferred_element_type=jnp.float32)
    for r in range(SR):
        out_ref[pl.ds(r * N, N)] = res[r, :N]


def kernel(x, edge_index, eps, W1, W_mu, W_logstd):
    E = edge_index.shape[1]
    src = edge_index[0].reshape(E // CH, CH)
    dst = edge_index[1].reshape(E // CH, CH)
    Wc = jnp.concatenate([W_mu, W_logstd], axis=1)
    Wbig = jnp.kron(jnp.eye(4, dtype=jnp.float32), Wc)
    zeros = jnp.zeros((NP, HF), jnp.float32)

    hw1 = pl.pallas_call(
        _mm1_body,
        out_shape=jax.ShapeDtypeStruct((NP, HF), jnp.float32),
    )(x, W1)

    segsum = _segsum_kernel(E)
    p1 = segsum(hw1, src, dst, zeros)

    hw2 = pl.pallas_call(
        _mm2_body,
        out_shape=jax.ShapeDtypeStruct((NPK, 128), jnp.float32),
    )(p1.reshape(NC, NPK, 128), Wbig)

    p2 = segsum(hw2.reshape(NP, HF), src, dst, zeros)

    zp = pl.pallas_call(
        _z_body,
        out_shape=jax.ShapeDtypeStruct((NP, 16), jnp.float32),
    )(p2, eps)

    recon = pl.pallas_call(
        _dec_body,
        grid=((N + SR - 1) // SR,),
        in_specs=[
            pl.BlockSpec((SR, 16), lambda i: (i, 0)),
            pl.BlockSpec((NP, 16), lambda i: (0, 0)),
        ],
        out_specs=pl.BlockSpec((SR * N,), lambda i: (i,)),
        out_shape=jax.ShapeDtypeStruct((N * N,), jnp.float32),
    )(zp, zp)

    return recon


# speedup vs baseline: 17.3670x; 1.0012x over previous
import functools

import jax
import jax.numpy as jnp
from jax import lax
from jax.experimental import pallas as pl
from jax.experimental.pallas import tpu as pltpu
from jax.experimental.pallas import tpu_sc as plsc

N = 10000
NP = 10240
HF = 32
NC, NS = 2, 16
NW = NC * NS
CH = 128
NBUF = 13


def _segsum_kernel(E):
    nrows = E // CH
    cpw = nrows // NW
    ntail = nrows - cpw * NW
    rps = NP // NS

    mesh = plsc.VectorSubcoreMesh(
        core_axis_name="c", subcore_axis_name="s", num_cores=NC, num_subcores=NS
    )

    @functools.partial(
        pl.kernel,
        out_type=jax.ShapeDtypeStruct((NC, NP, HF), jnp.float32),
        mesh=mesh,
        scratch_types=[
            pltpu.VMEM((cpw + 1, CH), jnp.int32),
            pltpu.VMEM((cpw + 1, CH), jnp.int32),
            [pltpu.VMEM((CH, HF), jnp.float32) for _ in range(NBUF)],
            pltpu.VMEM_SHARED((NP, HF), jnp.float32),
            [pltpu.SemaphoreType.DMA for _ in range(NBUF)],
            pltpu.SemaphoreType.DMA,
        ],
        compiler_params=pltpu.CompilerParams(use_tc_tiling_on_sc=False),
    )
    def seg(hw_hbm, src2_hbm, dst2_hbm, zeros_hbm, out_hbm,
            src_v, dst_v, rows_v, acc_sh, gsem, isem):
        c = lax.axis_index("c")
        s = lax.axis_index("s")
        wid = s * NC + c
        cbase = wid * cpw
        has_tail = wid < ntail

        pltpu.async_copy(src2_hbm.at[pl.ds(cbase, cpw)],
                         src_v.at[pl.ds(0, cpw)], isem)
        pltpu.async_copy(dst2_hbm.at[pl.ds(cbase, cpw)],
                         dst_v.at[pl.ds(0, cpw)], isem)

        @pl.when(has_tail)
        def _():
            t = NW * cpw + wid
            pltpu.async_copy(src2_hbm.at[pl.ds(t, 1)],
                             src_v.at[pl.ds(cpw, 1)], isem)
            pltpu.async_copy(dst2_hbm.at[pl.ds(t, 1)],
                             dst_v.at[pl.ds(cpw, 1)], isem)
            pltpu.make_async_copy(src2_hbm.at[pl.ds(t, 1)],
                                  src_v.at[pl.ds(cpw, 1)], isem).wait()
            pltpu.make_async_copy(dst2_hbm.at[pl.ds(t, 1)],
                                  dst_v.at[pl.ds(cpw, 1)], isem).wait()

        pltpu.make_async_copy(src2_hbm.at[pl.ds(cbase, cpw)],
                              src_v.at[pl.ds(0, cpw)], isem).wait()
        pltpu.make_async_copy(dst2_hbm.at[pl.ds(cbase, cpw)],
                              dst_v.at[pl.ds(0, cpw)], isem).wait()

        for b in range(NBUF):
            pltpu.async_copy(hw_hbm.at[src_v.at[b]], rows_v[b], gsem[b])

        pltpu.sync_copy(zeros_hbm.at[pl.ds(s * rps, rps)],
                        acc_sh.at[pl.ds(s * rps, rps)])
        plsc.subcore_barrier()

        def outer(io, carry):
            i0 = io * NBUF
            for b in range(NBUF):
                i = i0 + b
                pltpu.make_async_copy(hw_hbm.at[src_v.at[i]], rows_v[b],
                                      gsem[b]).wait()
                pltpu.sync_copy(rows_v[b], acc_sh.at[dst_v.at[i]], add=True)
                nxt = i + NBUF

                @pl.when(nxt < cpw)
                def _():
                    pltpu.async_copy(hw_hbm.at[src_v.at[nxt]], rows_v[b],
                                     gsem[b])
            return carry

        lax.fori_loop(0, cpw // NBUF, outer, 0)

        @pl.when(has_tail)
        def _():
            pltpu.async_copy(hw_hbm.at[src_v.at[cpw]], rows_v[0], gsem[0])
            pltpu.make_async_copy(hw_hbm.at[src_v.at[cpw]], rows_v[0],
                                  gsem[0]).wait()
            pltpu.sync_copy(rows_v[0], acc_sh.at[dst_v.at[cpw]], add=True)

        plsc.subcore_barrier()

        pltpu.sync_copy(acc_sh.at[pl.ds(s * rps, rps)],
                        out_hbm.at[c, pl.ds(s * rps, rps)])

    return seg


NPK = NP * HF // 128


def _mm1_body(x_ref, w_ref, out_ref):
    out_ref[pl.ds(0, N), :] = jnp.dot(x_ref[...], w_ref[...],
                                      preferred_element_type=jnp.float32)
    out_ref[pl.ds(N, NP - N), :] = jnp.zeros((NP - N, HF), jnp.float32)


def _mm2_body(p_ref, w_ref, out_ref):
    h = jnp.maximum(p_ref[0] + p_ref[1], 0.0)
    out_ref[...] = jnp.dot(h, w_ref[...], preferred_element_type=jnp.float32)


def _z_body(p_ref, eps_ref, out_ref):
    agg = p_ref[0] + p_ref[1]
    z = agg[:N, :16] + eps_ref[...] * jnp.exp(agg[:N, 16:])
    out_ref[pl.ds(0, N), :] = z
    out_ref[pl.ds(N, NP - N), :] = jnp.zeros((NP - N, 16), jnp.float32)


SR = 320


def _dec_body(zi_ref, zj_ref, out_ref):
    res = lax.dot_general(
        zi_ref[...], zj_ref[...], (((1,), (1,)), ((), ())),
        preferred_element_type=jnp.float32)
    for r in range(SR):
        out_ref[pl.ds(r * N, N)] = res[r, :N]


def kernel(x, edge_index, eps, W1, W_mu, W_logstd):
    E = edge_index.shape[1]
    src = edge_index[0].reshape(E // CH, CH)
    dst = edge_index[1].reshape(E // CH, CH)
    Wc = jnp.concatenate([W_mu, W_logstd], axis=1)
    Wbig = jnp.kron(jnp.eye(4, dtype=jnp.float32), Wc)
    zeros = jnp.zeros((NP, HF), jnp.float32)

    hw1 = pl.pallas_call(
        _mm1_body,
        out_shape=jax.ShapeDtypeStruct((NP, HF), jnp.float32),
    )(x, W1)

    segsum = _segsum_kernel(E)
    p1 = segsum(hw1, src, dst, zeros)

    hw2 = pl.pallas_call(
        _mm2_body,
        out_shape=jax.ShapeDtypeStruct((NPK, 128), jnp.float32),
    )(p1.reshape(NC, NPK, 128), Wbig)

    p2 = segsum(hw2.reshape(NP, HF), src, dst, zeros)

    zp = pl.pallas_call(
        _z_body,
        out_shape=jax.ShapeDtypeStruct((NP, 16), jnp.float32),
    )(p2, eps)

    recon = pl.pallas_call(
        _dec_body,
        grid=((N + SR - 1) // SR,),
        in_specs=[
            pl.BlockSpec((SR, 16), lambda i: (i, 0)),
            pl.BlockSpec((NP, 16), lambda i: (0, 0)),
        ],
        out_specs=pl.BlockSpec((SR * N,), lambda i: (i,)),
        out_shape=jax.ShapeDtypeStruct((N * N,), jnp.float32),
    )(zp, zp)

    return recon
